# baseline (speedup 1.0000x reference)
import functools

import jax
import jax.numpy as jnp
from jax import lax
from jax.experimental import pallas as pl
from jax.experimental.pallas import tpu as pltpu
from jax.experimental.pallas import tpu_sc as plsc

N = 10000
E = 320000
H = 128
G = 64

NC = 2
NS = 16
NW = NC * NS

C = 128
CH = 160
TOTCH = NS * CH
NEP = TOTCH * C
DUMMY = N

ZPT = 640
ZR = NS * ZPT


def _sc_body(h_hbm, eidx_hbm, out_hbm,
             agg_sp, idx0, idx1, rows0, rows1, si0, si1, sg0, sg1):
    cid = lax.axis_index("c")
    sid = lax.axis_index("s")

    @pl.when(cid == 0)
    def _core0_work():
        cbase = sid * CH
        zero = jnp.zeros((16,), jnp.float32)

        @pl.loop(0, C)
        def _zb(r):
            for c0 in range(H // 16):
                rows0[r, pl.ds(c0 * 16, 16)] = zero

        @pl.loop(0, ZPT // C)
        def _zero(j):
            pltpu.sync_copy(rows0, agg_sp.at[pl.ds(sid * ZPT + j * C, C)])

        plsc.subcore_barrier()

        def istart(jj, ibuf, sem):
            pltpu.async_copy(eidx_hbm.at[cbase + jj], ibuf, sem)

        def iwait(ibuf, sem):
            pltpu.make_async_copy(eidx_hbm.at[0], ibuf, sem).wait()

        def gstart(ibuf, buf, sem):
            pltpu.async_copy(h_hbm.at[ibuf.at[0]], buf, sem)

        def gwait(buf, sem):
            pltpu.make_async_copy(h_hbm.at[pl.ds(0, C)], buf, sem).wait()

        def scat(ibuf, buf):
            pltpu.sync_copy(buf, agg_sp.at[ibuf.at[1]], add=True)

        istart(0, idx0, si0)
        istart(1, idx1, si1)
        iwait(idx0, si0)
        gstart(idx0, rows0, sg0)

        @pl.loop(0, CH - 2, step=2)
        def _edges(j):
            iwait(idx1, si1)
            gstart(idx1, rows1, sg1)
            gwait(rows0, sg0)
            scat(idx0, rows0)
            istart(j + 2, idx0, si0)

            iwait(idx0, si0)
            gstart(idx0, rows0, sg0)
            gwait(rows1, sg1)
            scat(idx1, rows1)
            istart(j + 3, idx1, si1)

        iwait(idx1, si1)
        gstart(idx1, rows1, sg1)
        gwait(rows0, sg0)
        scat(idx0, rows0)
        gwait(rows1, sg1)
        scat(idx1, rows1)

        plsc.subcore_barrier()

        pltpu.sync_copy(agg_sp.at[pl.ds(sid * ZPT, ZPT)],
                        out_hbm.at[pl.ds(sid * ZPT, ZPT)])


@functools.cache
def _get_sc_aggregate():
    mesh = plsc.VectorSubcoreMesh(core_axis_name="c", subcore_axis_name="s",
                                  num_cores=NC, num_subcores=NS)
    return pl.kernel(
        _sc_body,
        out_type=jax.ShapeDtypeStruct((ZR, H), jnp.float32),
        mesh=mesh,
        scratch_types=[
            pltpu.VMEM_SHARED((ZR, H), jnp.float32),
            pltpu.VMEM((2, C), jnp.int32),
            pltpu.VMEM((2, C), jnp.int32),
            pltpu.VMEM((C, H), jnp.float32),
            pltpu.VMEM((C, H), jnp.float32),
            pltpu.SemaphoreType.DMA,
            pltpu.SemaphoreType.DMA,
            pltpu.SemaphoreType.DMA,
            pltpu.SemaphoreType.DMA,
        ],
    )


BR = 2000
NB = N // BR


def _layer_body(agg_ref, h_ref, wrel_ref, brel_ref, wroot_ref, out_ref):
    acc = jnp.dot(agg_ref[...], wrel_ref[...], preferred_element_type=jnp.float32)
    acc += jnp.dot(h_ref[...], wroot_ref[...], preferred_element_type=jnp.float32)
    acc += brel_ref[...]
    out_ref[...] = jnp.maximum(acc, 0.0)


_layer_call = pl.pallas_call(
    _layer_body,
    grid=(NB,),
    in_specs=[
        pl.BlockSpec((BR, H), lambda i: (i, 0)),
        pl.BlockSpec((BR, H), lambda i: (i, 0)),
        pl.BlockSpec((H, H), lambda i: (0, 0)),
        pl.BlockSpec((1, H), lambda i: (0, 0)),
        pl.BlockSpec((H, H), lambda i: (0, 0)),
    ],
    out_specs=pl.BlockSpec((BR, H), lambda i: (i, 0)),
    out_shape=jax.ShapeDtypeStruct((N, H), jnp.float32),
)


def _pool_body(h1_ref, h2_ref, h3_ref, batch_ref,
               w1_ref, b1_ref, w2_ref, b2_ref, w3_ref, b3_ref,
               out_ref, pacc):
    pid = pl.program_id(0)

    @pl.when(pid == 0)
    def _():
        pacc[...] = jnp.zeros_like(pacc)

    hsum = h1_ref[...] + h2_ref[...] + h3_ref[...]
    b = batch_ref[0, 0, :]
    onehot = (b[:, None] == lax.broadcasted_iota(jnp.int32, (1, G), 1)
              ).astype(jnp.float32)
    pacc[...] += lax.dot_general(onehot, hsum, (((0,), (0,)), ((), ())),
                                 precision=lax.Precision.HIGHEST,
                                 preferred_element_type=jnp.float32)

    o = jnp.maximum(
        jnp.dot(pacc[...], w1_ref[...], preferred_element_type=jnp.float32)
        + b1_ref[...], 0.0)
    o = jnp.maximum(
        jnp.dot(o, w2_ref[...], preferred_element_type=jnp.float32)
        + b2_ref[...], 0.0)
    out_ref[...] = (jnp.dot(o, w3_ref[...], preferred_element_type=jnp.float32)
                    + b3_ref[...])


_pool_call = pl.pallas_call(
    _pool_body,
    grid=(NB,),
    in_specs=[
        pl.BlockSpec((BR, H), lambda i: (i, 0)),
        pl.BlockSpec((BR, H), lambda i: (i, 0)),
        pl.BlockSpec((BR, H), lambda i: (i, 0)),
        pl.BlockSpec((1, 1, BR), lambda i: (i, 0, 0)),
        pl.BlockSpec((H, 64), lambda i: (0, 0)),
        pl.BlockSpec((1, 64), lambda i: (0, 0)),
        pl.BlockSpec((64, 32), lambda i: (0, 0)),
        pl.BlockSpec((1, 32), lambda i: (0, 0)),
        pl.BlockSpec((32, 1), lambda i: (0, 0)),
        pl.BlockSpec((1, 1), lambda i: (0, 0)),
    ],
    out_specs=pl.BlockSpec((G, 1), lambda i: (0, 0)),
    out_shape=jax.ShapeDtypeStruct((G, 1), jnp.float32),
    scratch_shapes=[pltpu.VMEM((G, H), jnp.float32)],
)


def kernel(x, edge_index, batch, W_rel, b_rel, W_root, W1, b1, W2, b2, W3, b3):
    src = edge_index[0].astype(jnp.int32)
    dst = edge_index[1].astype(jnp.int32)
    pad = NEP - E
    src_p = jnp.concatenate([src, jnp.zeros((pad,), jnp.int32)]).reshape(TOTCH, C)
    dst_p = jnp.concatenate([dst, jnp.full((pad,), DUMMY, jnp.int32)]).reshape(TOTCH, C)
    eidx = jnp.stack([src_p, dst_p], axis=1)
    batch3 = batch.astype(jnp.int32).reshape(NB, 1, BR)

    h = x
    hs = []
    for i in range(3):
        agg2 = _get_sc_aggregate()(h, eidx)
        h = _layer_call(agg2, h, W_rel[i], b_rel[i].reshape(1, H), W_root[i])
        hs.append(h)

    return _pool_call(hs[0], hs[1], hs[2], batch3,
                      W1, b1.reshape(1, 64), W2, b2.reshape(1, 32),
                      W3, b3.reshape(1, 1))


# speedup vs baseline: 3.5512x; 3.5512x over previous
import functools

import jax
import jax.numpy as jnp
from jax import lax
from jax.experimental import pallas as pl
from jax.experimental.pallas import tpu as pltpu
from jax.experimental.pallas import tpu_sc as plsc

N = 10000
E = 320000
H = 128
G = 64

NC = 2
NS = 16
NW = NC * NS

C = 128
CH = 80
TOTCH = NW * CH
NEP = TOTCH * C

ZPT = 640
ZR = NS * ZPT


def _sc_body(h_hbm, eidx_hbm, out_hbm,
             agg_sp, idx0, idx1, rows0, rows1, si0, si1, sg0, sg1):
    cid = lax.axis_index("c")
    sid = lax.axis_index("s")
    cbase = (cid * NS + sid) * CH
    zero = jnp.zeros((16,), jnp.float32)

    @pl.loop(0, C)
    def _zb(r):
        for c0 in range(H // 16):
            rows0[r, pl.ds(c0 * 16, 16)] = zero

    @pl.loop(0, ZPT // C)
    def _zero(j):
        pltpu.sync_copy(rows0, agg_sp.at[pl.ds(sid * ZPT + j * C, C)])

    plsc.subcore_barrier()

    def istart(jj, ibuf, sem):
        pltpu.async_copy(eidx_hbm.at[cbase + jj], ibuf, sem)

    def iwait(ibuf, sem):
        pltpu.make_async_copy(eidx_hbm.at[0], ibuf, sem).wait()

    def gstart(ibuf, buf, sem):
        pltpu.async_copy(h_hbm.at[ibuf.at[0]], buf, sem)

    def gwait(buf, sem):
        pltpu.make_async_copy(h_hbm.at[pl.ds(0, C)], buf, sem).wait()

    def scat(ibuf, buf):
        pltpu.sync_copy(buf, agg_sp.at[ibuf.at[1]], add=True)

    istart(0, idx0, si0)
    istart(1, idx1, si1)
    iwait(idx0, si0)
    gstart(idx0, rows0, sg0)

    @pl.loop(0, CH - 2, step=2)
    def _edges(j):
        iwait(idx1, si1)
        gstart(idx1, rows1, sg1)
        gwait(rows0, sg0)
        scat(idx0, rows0)
        istart(j + 2, idx0, si0)

        iwait(idx0, si0)
        gstart(idx0, rows0, sg0)
        gwait(rows1, sg1)
        scat(idx1, rows1)
        istart(j + 3, idx1, si1)

    iwait(idx1, si1)
    gstart(idx1, rows1, sg1)
    gwait(rows0, sg0)
    scat(idx0, rows0)
    gwait(rows1, sg1)
    scat(idx1, rows1)

    plsc.subcore_barrier()

    pltpu.sync_copy(agg_sp.at[pl.ds(sid * ZPT, ZPT)],
                    out_hbm.at[cid, pl.ds(sid * ZPT, ZPT)])


@functools.cache
def _get_sc_aggregate():
    mesh = plsc.VectorSubcoreMesh(core_axis_name="c", subcore_axis_name="s",
                                  num_cores=NC, num_subcores=NS)
    return pl.kernel(
        _sc_body,
        out_type=jax.ShapeDtypeStruct((NC, ZR, H), jnp.float32),
        mesh=mesh,
        scratch_types=[
            pltpu.VMEM_SHARED((ZR, H), jnp.float32),
            pltpu.VMEM((2, C), jnp.int32),
            pltpu.VMEM((2, C), jnp.int32),
            pltpu.VMEM((C, H), jnp.float32),
            pltpu.VMEM((C, H), jnp.float32),
            pltpu.SemaphoreType.DMA,
            pltpu.SemaphoreType.DMA,
            pltpu.SemaphoreType.DMA,
            pltpu.SemaphoreType.DMA,
        ],
    )


BR = 2000
NB = N // BR


def _layer_body(agg_ref, h_ref, wrel_ref, brel_ref, wroot_ref, out_ref):
    aggsum = agg_ref[0] + agg_ref[1]
    acc = jnp.dot(aggsum, wrel_ref[...], preferred_element_type=jnp.float32)
    acc += jnp.dot(h_ref[...], wroot_ref[...], preferred_element_type=jnp.float32)
    acc += brel_ref[...]
    out_ref[...] = jnp.maximum(acc, 0.0)


_layer_call = pl.pallas_call(
    _layer_body,
    grid=(NB,),
    in_specs=[
        pl.BlockSpec((NC, BR, H), lambda i: (0, i, 0)),
        pl.BlockSpec((BR, H), lambda i: (i, 0)),
        pl.BlockSpec((H, H), lambda i: (0, 0)),
        pl.BlockSpec((1, H), lambda i: (0, 0)),
        pl.BlockSpec((H, H), lambda i: (0, 0)),
    ],
    out_specs=pl.BlockSpec((BR, H), lambda i: (i, 0)),
    out_shape=jax.ShapeDtypeStruct((N, H), jnp.float32),
)


def _pool_body(h1_ref, h2_ref, h3_ref, batch_ref,
               w1_ref, b1_ref, w2_ref, b2_ref, w3_ref, b3_ref,
               out_ref, pacc):
    pid = pl.program_id(0)

    @pl.when(pid == 0)
    def _():
        pacc[...] = jnp.zeros_like(pacc)

    hsum = h1_ref[...] + h2_ref[...] + h3_ref[...]
    b = batch_ref[0, 0, :]
    onehot = (b[:, None] == lax.broadcasted_iota(jnp.int32, (1, G), 1)
              ).astype(jnp.float32)
    pacc[...] += lax.dot_general(onehot, hsum, (((0,), (0,)), ((), ())),
                                 precision=lax.Precision.HIGHEST,
                                 preferred_element_type=jnp.float32)

    o = jnp.maximum(
        jnp.dot(pacc[...], w1_ref[...], preferred_element_type=jnp.float32)
        + b1_ref[...], 0.0)
    o = jnp.maximum(
        jnp.dot(o, w2_ref[...], preferred_element_type=jnp.float32)
        + b2_ref[...], 0.0)
    out_ref[...] = (jnp.dot(o, w3_ref[...], preferred_element_type=jnp.float32)
                    + b3_ref[...])


_pool_call = pl.pallas_call(
    _pool_body,
    grid=(NB,),
    in_specs=[
        pl.BlockSpec((BR, H), lambda i: (i, 0)),
        pl.BlockSpec((BR, H), lambda i: (i, 0)),
        pl.BlockSpec((BR, H), lambda i: (i, 0)),
        pl.BlockSpec((1, 1, BR), lambda i: (i, 0, 0)),
        pl.BlockSpec((H, 64), lambda i: (0, 0)),
        pl.BlockSpec((1, 64), lambda i: (0, 0)),
        pl.BlockSpec((64, 32), lambda i: (0, 0)),
        pl.BlockSpec((1, 32), lambda i: (0, 0)),
        pl.BlockSpec((32, 1), lambda i: (0, 0)),
        pl.BlockSpec((1, 1), lambda i: (0, 0)),
    ],
    out_specs=pl.BlockSpec((G, 1), lambda i: (0, 0)),
    out_shape=jax.ShapeDtypeStruct((G, 1), jnp.float32),
    scratch_shapes=[pltpu.VMEM((G, H), jnp.float32)],
)


def kernel(x, edge_index, batch, W_rel, b_rel, W_root, W1, b1, W2, b2, W3, b3):
    src = edge_index[0].astype(jnp.int32)
    dst = edge_index[1].astype(jnp.int32)
    pad = NEP - E
    pad_src = jnp.arange(pad, dtype=jnp.int32) % N
    pad_dst = N + (jnp.arange(pad, dtype=jnp.int32) % (ZR - N))
    src_p = jnp.concatenate([src, pad_src]).reshape(TOTCH, C)
    dst_p = jnp.concatenate([dst, pad_dst]).reshape(TOTCH, C)
    eidx = jnp.stack([src_p, dst_p], axis=1)
    batch3 = batch.astype(jnp.int32).reshape(NB, 1, BR)

    h = x
    hs = []
    for i in range(3):
        agg2 = _get_sc_aggregate()(h, eidx)
        h = _layer_call(agg2, h, W_rel[i], b_rel[i].reshape(1, H), W_root[i])
        hs.append(h)

    return _pool_call(hs[0], hs[1], hs[2], batch3,
                      W1, b1.reshape(1, 64), W2, b2.reshape(1, 32),
                      W3, b3.reshape(1, 1))


# speedup vs baseline: 3.5960x; 1.0126x over previous
import functools

import jax
import jax.numpy as jnp
from jax import lax
from jax.experimental import pallas as pl
from jax.experimental.pallas import tpu as pltpu
from jax.experimental.pallas import tpu_sc as plsc

N = 10000
E = 320000
H = 128
G = 64

NC = 2
NS = 16
NW = NC * NS

C = 128
CH = 80
TOTCH = NW * CH
NEP = TOTCH * C

ZPT = 640
ZR = NS * ZPT


def _sc_body(h_hbm, eidx_hbm, out_hbm,
             agg_sp, idx0, idx1, rows0, rows1, si0, si1, sg0, sg1):
    cid = lax.axis_index("c")
    sid = lax.axis_index("s")
    cbase = (cid * NS + sid) * CH
    zero = jnp.zeros((16,), jnp.float32)

    @pl.loop(0, C)
    def _zb(r):
        for c0 in range(H // 16):
            rows0[r, pl.ds(c0 * 16, 16)] = zero

    @pl.loop(0, ZPT // C)
    def _zero(j):
        pltpu.sync_copy(rows0, agg_sp.at[pl.ds(sid * ZPT + j * C, C)])

    plsc.subcore_barrier()

    def istart(jj, ibuf, sem):
        pltpu.async_copy(eidx_hbm.at[cbase + jj], ibuf, sem)

    def iwait(ibuf, sem):
        pltpu.make_async_copy(eidx_hbm.at[0], ibuf, sem).wait()

    def gstart(ibuf, buf, sem):
        pltpu.async_copy(h_hbm.at[ibuf.at[0]], buf, sem)

    def gwait(buf, sem):
        pltpu.make_async_copy(h_hbm.at[pl.ds(0, C)], buf, sem).wait()

    def scat(ibuf, buf):
        pltpu.sync_copy(buf, agg_sp.at[ibuf.at[1]], add=True)

    istart(0, idx0, si0)
    istart(1, idx1, si1)
    iwait(idx0, si0)
    gstart(idx0, rows0, sg0)

    @pl.loop(0, CH - 2, step=2)
    def _edges(j):
        iwait(idx1, si1)
        gstart(idx1, rows1, sg1)
        gwait(rows0, sg0)
        scat(idx0, rows0)
        istart(j + 2, idx0, si0)

        iwait(idx0, si0)
        gstart(idx0, rows0, sg0)
        gwait(rows1, sg1)
        scat(idx1, rows1)
        istart(j + 3, idx1, si1)

    iwait(idx1, si1)
    gstart(idx1, rows1, sg1)
    gwait(rows0, sg0)
    scat(idx0, rows0)
    gwait(rows1, sg1)
    scat(idx1, rows1)

    plsc.subcore_barrier()

    pltpu.sync_copy(agg_sp.at[pl.ds(sid * ZPT, ZPT)],
                    out_hbm.at[cid, pl.ds(sid * ZPT, ZPT)])


@functools.cache
def _get_sc_aggregate():
    mesh = plsc.VectorSubcoreMesh(core_axis_name="c", subcore_axis_name="s",
                                  num_cores=NC, num_subcores=NS)
    return pl.kernel(
        _sc_body,
        out_type=jax.ShapeDtypeStruct((NC, ZR, H), jnp.float32),
        mesh=mesh,
        scratch_types=[
            pltpu.VMEM_SHARED((ZR, H), jnp.float32),
            pltpu.VMEM((2, C), jnp.int32),
            pltpu.VMEM((2, C), jnp.int32),
            pltpu.VMEM((C, H), jnp.float32),
            pltpu.VMEM((C, H), jnp.float32),
            pltpu.SemaphoreType.DMA,
            pltpu.SemaphoreType.DMA,
            pltpu.SemaphoreType.DMA,
            pltpu.SemaphoreType.DMA,
        ],
    )


BR = 2000
NB = N // BR


def _layer_pool_body(agg_ref, h_ref, wrel_ref, brel_ref, wroot_ref,
                     batch_ref, pin_ref, hout_ref, pout_ref):
    pid = pl.program_id(0)
    aggsum = agg_ref[0] + agg_ref[1]
    acc = jnp.dot(aggsum, wrel_ref[...], preferred_element_type=jnp.float32)
    acc += jnp.dot(h_ref[...], wroot_ref[...], preferred_element_type=jnp.float32)
    acc += brel_ref[...]
    hn = jnp.maximum(acc, 0.0)
    hout_ref[...] = hn

    @pl.when(pid == 0)
    def _():
        pout_ref[...] = pin_ref[...]

    b = batch_ref[0, 0, :]
    onehot = (b[:, None] == lax.broadcasted_iota(jnp.int32, (1, G), 1)
              ).astype(jnp.float32)
    pout_ref[...] += lax.dot_general(onehot, hn, (((0,), (0,)), ((), ())),
                                     precision=lax.Precision.HIGHEST,
                                     preferred_element_type=jnp.float32)


_layer_pool_call = pl.pallas_call(
    _layer_pool_body,
    grid=(NB,),
    in_specs=[
        pl.BlockSpec((NC, BR, H), lambda i: (0, i, 0)),
        pl.BlockSpec((BR, H), lambda i: (i, 0)),
        pl.BlockSpec((H, H), lambda i: (0, 0)),
        pl.BlockSpec((1, H), lambda i: (0, 0)),
        pl.BlockSpec((H, H), lambda i: (0, 0)),
        pl.BlockSpec((1, 1, BR), lambda i: (i, 0, 0)),
        pl.BlockSpec((G, H), lambda i: (0, 0)),
    ],
    out_specs=[
        pl.BlockSpec((BR, H), lambda i: (i, 0)),
        pl.BlockSpec((G, H), lambda i: (0, 0)),
    ],
    out_shape=[
        jax.ShapeDtypeStruct((N, H), jnp.float32),
        jax.ShapeDtypeStruct((G, H), jnp.float32),
    ],
)


def _layer3_body(agg_ref, h_ref, wrel_ref, brel_ref, wroot_ref,
                 batch_ref, pin_ref,
                 w1_ref, b1_ref, w2_ref, b2_ref, w3_ref, b3_ref,
                 out_ref, pacc):
    pid = pl.program_id(0)
    aggsum = agg_ref[0] + agg_ref[1]
    acc = jnp.dot(aggsum, wrel_ref[...], preferred_element_type=jnp.float32)
    acc += jnp.dot(h_ref[...], wroot_ref[...], preferred_element_type=jnp.float32)
    acc += brel_ref[...]
    hn = jnp.maximum(acc, 0.0)

    @pl.when(pid == 0)
    def _():
        pacc[...] = pin_ref[...]

    b = batch_ref[0, 0, :]
    onehot = (b[:, None] == lax.broadcasted_iota(jnp.int32, (1, G), 1)
              ).astype(jnp.float32)
    pacc[...] += lax.dot_general(onehot, hn, (((0,), (0,)), ((), ())),
                                 precision=lax.Precision.HIGHEST,
                                 preferred_element_type=jnp.float32)

    o = jnp.maximum(
        jnp.dot(pacc[...], w1_ref[...], preferred_element_type=jnp.float32)
        + b1_ref[...], 0.0)
    o = jnp.maximum(
        jnp.dot(o, w2_ref[...], preferred_element_type=jnp.float32)
        + b2_ref[...], 0.0)
    out_ref[...] = (jnp.dot(o, w3_ref[...], preferred_element_type=jnp.float32)
                    + b3_ref[...])


_layer3_call = pl.pallas_call(
    _layer3_body,
    grid=(NB,),
    in_specs=[
        pl.BlockSpec((NC, BR, H), lambda i: (0, i, 0)),
        pl.BlockSpec((BR, H), lambda i: (i, 0)),
        pl.BlockSpec((H, H), lambda i: (0, 0)),
        pl.BlockSpec((1, H), lambda i: (0, 0)),
        pl.BlockSpec((H, H), lambda i: (0, 0)),
        pl.BlockSpec((1, 1, BR), lambda i: (i, 0, 0)),
        pl.BlockSpec((G, H), lambda i: (0, 0)),
        pl.BlockSpec((H, 64), lambda i: (0, 0)),
        pl.BlockSpec((1, 64), lambda i: (0, 0)),
        pl.BlockSpec((64, 32), lambda i: (0, 0)),
        pl.BlockSpec((1, 32), lambda i: (0, 0)),
        pl.BlockSpec((32, 1), lambda i: (0, 0)),
        pl.BlockSpec((1, 1), lambda i: (0, 0)),
    ],
    out_specs=pl.BlockSpec((G, 1), lambda i: (0, 0)),
    out_shape=jax.ShapeDtypeStruct((G, 1), jnp.float32),
    scratch_shapes=[pltpu.VMEM((G, H), jnp.float32)],
)


def kernel(x, edge_index, batch, W_rel, b_rel, W_root, W1, b1, W2, b2, W3, b3):
    src = edge_index[0].astype(jnp.int32)
    dst = edge_index[1].astype(jnp.int32)
    pad = NEP - E
    pad_src = jnp.arange(pad, dtype=jnp.int32) % N
    pad_dst = N + (jnp.arange(pad, dtype=jnp.int32) % (ZR - N))
    src_p = jnp.concatenate([src, pad_src]).reshape(TOTCH, C)
    dst_p = jnp.concatenate([dst, pad_dst]).reshape(TOTCH, C)
    eidx = jnp.stack([src_p, dst_p], axis=1)
    batch3 = batch.astype(jnp.int32).reshape(NB, 1, BR)

    h = x
    p = jnp.zeros((G, H), jnp.float32)
    for i in range(2):
        agg2 = _get_sc_aggregate()(h, eidx)
        h, p = _layer_pool_call(agg2, h, W_rel[i], b_rel[i].reshape(1, H),
                                W_root[i], batch3, p)
    agg2 = _get_sc_aggregate()(h, eidx)
    return _layer3_call(agg2, h, W_rel[2], b_rel[2].reshape(1, H), W_root[2],
                        batch3, p,
                        W1, b1.reshape(1, 64), W2, b2.reshape(1, 32),
                        W3, b3.reshape(1, 1))
